# baseline (speedup 1.0000x reference)
import functools

import jax
import jax.numpy as jnp
from jax import lax
from jax.experimental import pallas as pl
from jax.experimental.pallas import tpu as pltpu
from jax.experimental.pallas import tpu_sc as plsc

N = 10000
E = 320000
D = 128

NC = 2
NS = 16
NW = NC * NS
CH = 128
R2D = 2560
EPAD = R2D * CH
RPC = R2D // NW
CHS = 128
EPW = EPAD // NW
NCHS = EPW // CHS
NBUF = 2
NPAD = 10240
RPT = NPAD // NS
DPT = NPAD // NS

_sc_mesh = plsc.VectorSubcoreMesh(core_axis_name="c", subcore_axis_name="s")


@functools.partial(
    pl.kernel,
    out_type=[jax.ShapeDtypeStruct((NC * NPAD,), jnp.float32)] * 4,
    mesh=_sc_mesh,
    scratch_types=[
        pltpu.VMEM((RPC, CH), jnp.int32),
        pltpu.VMEM((RPC, CH), jnp.int32),
        pltpu.VMEM((RPC, CH), jnp.int32),
        pltpu.VMEM((RPC, CH), jnp.int32),
        pltpu.VMEM((CH,), jnp.float32),
        pltpu.VMEM_SHARED((NPAD,), jnp.float32),
        pltpu.VMEM_SHARED((NPAD,), jnp.float32),
        pltpu.VMEM_SHARED((NPAD,), jnp.float32),
        pltpu.VMEM_SHARED((NPAD,), jnp.float32),
        pltpu.SemaphoreType.DMA,
        pltpu.SemaphoreType.DMA,
        pltpu.SemaphoreType.DMA,
        pltpu.SemaphoreType.DMA,
    ],
)
def _sc_degrees(s1_hbm, d1_hbm, s2_hbm, d2_hbm, ones_hbm, zeros_hbm,
                o1_hbm, i1_hbm, o2_hbm, i2_hbm,
                s1_v, d1_v, s2_v, d2_v, ones_v,
                o1_sh, i1_sh, o2_sh, i2_sh,
                sem0, sem1, sem2, sem3):
    cid = lax.axis_index("c")
    sid = lax.axis_index("s")
    wid = cid * NS + sid
    idx_v = (s1_v, d1_v, s2_v, d2_v)
    idx_hbm = (s1_hbm, d1_hbm, s2_hbm, d2_hbm)
    sh = (o1_sh, i1_sh, o2_sh, i2_sh)
    out = (o1_hbm, i1_hbm, o2_hbm, i2_hbm)
    sems = (sem0, sem1, sem2, sem3)

    pltpu.sync_copy(ones_hbm, ones_v)
    for k in range(4):
        pltpu.sync_copy(idx_hbm[k].at[pl.ds(wid * RPC, RPC)], idx_v[k])
        pltpu.sync_copy(zeros_hbm, sh[k].at[pl.ds(sid * DPT, DPT)])
    plsc.subcore_barrier()

    for k in range(4):
        pltpu.async_copy(ones_v, sh[k].at[idx_v[k].at[0]], sems[k], add=True)

    def body(j, carry):
        for k in range(4):
            pltpu.make_async_copy(ones_v, sh[k].at[idx_v[k].at[j - 1]],
                                  sems[k]).wait()
            pltpu.async_copy(ones_v, sh[k].at[idx_v[k].at[j]], sems[k],
                             add=True)
        return carry

    lax.fori_loop(1, RPC, body, 0)
    for k in range(4):
        pltpu.make_async_copy(ones_v, sh[k].at[idx_v[k].at[RPC - 1]],
                              sems[k]).wait()
    plsc.subcore_barrier()
    for k in range(4):
        pltpu.sync_copy(sh[k].at[pl.ds(sid * DPT, DPT)],
                        out[k].at[pl.ds(cid * NPAD + sid * DPT, DPT)])


@functools.partial(
    pl.kernel,
    out_type=jax.ShapeDtypeStruct((NC, NPAD, D), jnp.float32),
    mesh=_sc_mesh,
    scratch_types=[
        [pltpu.VMEM((CHS,), jnp.int32)] * NBUF,
        [pltpu.VMEM((CHS,), jnp.int32)] * NBUF,
        [pltpu.VMEM((CHS, D), jnp.float32)] * NBUF,
        pltpu.VMEM_SHARED((NPAD, D), jnp.float32),
        [pltpu.SemaphoreType.DMA] * NBUF,
        [pltpu.SemaphoreType.DMA] * NBUF,
    ],
)
def _sc_spmm(h_hbm, src_hbm, dst_hbm, zrows_hbm, out_hbm,
             sidx, didx, rows, acc_sh, sg, ss):
    cid = lax.axis_index("c")
    sid = lax.axis_index("s")
    wid = cid * NS + sid
    base = wid * EPW

    pltpu.sync_copy(zrows_hbm, acc_sh.at[pl.ds(sid * RPT, RPT)])
    plsc.subcore_barrier()

    def body(j, carry):
        off = base + NBUF * j * CHS
        for p in range(NBUF):
            pltpu.sync_copy(src_hbm.at[pl.ds(off + p * CHS, CHS)], sidx[p])
            pltpu.sync_copy(dst_hbm.at[pl.ds(off + p * CHS, CHS)], didx[p])
        gd = [pltpu.async_copy(h_hbm.at[sidx[p]], rows[p], sg[p])
              for p in range(NBUF)]
        sd = []
        for p in range(NBUF):
            gd[p].wait()
            sd.append(pltpu.async_copy(rows[p], acc_sh.at[didx[p]], ss[p],
                                       add=True))
        for p in range(NBUF):
            sd[p].wait()
        return carry

    lax.fori_loop(0, NCHS // NBUF, body, 0)
    plsc.subcore_barrier()
    pltpu.sync_copy(acc_sh.at[pl.ds(sid * RPT, RPT)],
                    out_hbm.at[cid, pl.ds(sid * RPT, RPT)])


_BN = 2000


def _prep_body(feat_ref, doa_ref, dob_ref, dia_ref, dib_ref,
               h0_ref, ns_ref, nd_ref):
    dego = doa_ref[...] + dob_ref[...]
    degi = dia_ref[...] + dib_ref[...]
    ns = lax.rsqrt(jnp.where(dego > 0, dego, 1.0))
    nd = lax.rsqrt(jnp.where(degi > 0, degi, 1.0))
    ns_ref[...] = ns
    nd_ref[...] = nd
    h0_ref[...] = feat_ref[...] * ns


def _tc_prep(feat, doa, dob, dia, dib):
    grid = (N // _BN,)
    row = pl.BlockSpec((_BN, D), lambda i: (i, 0))
    col = pl.BlockSpec((_BN, 1), lambda i: (i, 0))
    return pl.pallas_call(
        _prep_body,
        grid=grid,
        in_specs=[row, col, col, col, col],
        out_specs=[row, col, col],
        out_shape=[
            jax.ShapeDtypeStruct((N, D), jnp.float32),
            jax.ShapeDtypeStruct((N, 1), jnp.float32),
            jax.ShapeDtypeStruct((N, 1), jnp.float32),
        ],
    )(feat, doa, dob, dia, dib)


def _mm_body(aa_ref, ab_ref, nd_ref, so_ref, w_ref, b_ref, y_ref):
    x = (aa_ref[0] + ab_ref[0]) * nd_ref[...]
    y = jnp.dot(x, w_ref[...], preferred_element_type=jnp.float32)
    y = jnp.maximum(y + b_ref[...], 0.0)
    y_ref[...] = y * so_ref[...]


def _tc_mm(aggp, nd, so, w, b):
    grid = (N // _BN,)
    part_a = pl.BlockSpec((1, _BN, D), lambda i: (0, i, 0))
    part_b = pl.BlockSpec((1, _BN, D), lambda i: (1, i, 0))
    row = pl.BlockSpec((_BN, D), lambda i: (i, 0))
    col = pl.BlockSpec((_BN, 1), lambda i: (i, 0))
    full = pl.BlockSpec((D, D), lambda i: (0, 0))
    vec = pl.BlockSpec((1, D), lambda i: (0, 0))
    return pl.pallas_call(
        _mm_body,
        grid=grid,
        in_specs=[part_a, part_b, col, col, full, vec],
        out_specs=row,
        out_shape=jax.ShapeDtypeStruct((N, D), jnp.float32),
    )(aggp, aggp, nd, so, w, b)


def _mlp_body(c_ref, w1_ref, b1_ref, w2_ref, b2_ref, y_ref):
    t = jnp.dot(c_ref[...], w1_ref[...], preferred_element_type=jnp.float32)
    t = jnp.maximum(t + b1_ref[...], 0.0)
    y = jnp.dot(t, w2_ref[...], preferred_element_type=jnp.float32)
    y_ref[...] = jnp.maximum(y + b2_ref[...], 0.0)


def _tc_mlp(c, w1, b1, w2, b2):
    grid = (N // _BN,)
    row = pl.BlockSpec((_BN, D), lambda i: (i, 0))
    full = pl.BlockSpec((D, D), lambda i: (0, 0))
    vec = pl.BlockSpec((1, D), lambda i: (0, 0))
    return pl.pallas_call(
        _mlp_body,
        grid=grid,
        in_specs=[row, full, vec, full, vec],
        out_specs=row,
        out_shape=jax.ShapeDtypeStruct((N, D), jnp.float32),
    )(c, w1, b1, w2, b2)


def _pad_edges(idx, pad_value):
    pad = jnp.full((EPAD - E,), pad_value, jnp.int32)
    return jnp.concatenate([idx, pad])


def _gcn_encode(feat, src_mm, dst_mm, dego, degi, W1, b1, W2, b2, zrows):
    doa = dego[:N].reshape(N, 1)
    dob = dego[NPAD:NPAD + N].reshape(N, 1)
    dia = degi[:N].reshape(N, 1)
    dib = degi[NPAD:NPAD + N].reshape(N, 1)

    h0, ns, nd = _tc_prep(feat, doa, dob, dia, dib)

    agg1 = _sc_spmm(h0, src_mm, dst_mm, zrows)
    h1 = _tc_mm(agg1, nd, ns, W1, b1.reshape(1, D))

    agg2 = _sc_spmm(h1, src_mm, dst_mm, zrows)
    ones_n = jnp.ones((N, 1), jnp.float32)
    c = _tc_mm(agg2, nd, ones_n, W2, b2.reshape(1, D))
    return c


def kernel(feat1, feat2, edge_index1, edge_index2, W1, b1, W2, b2,
           Wm1, bm1, Wm2, bm2):
    s1, d1 = edge_index1[0], edge_index1[1]
    s2, d2 = edge_index2[0], edge_index2[1]
    s1_mm = _pad_edges(s1, 0)
    s2_mm = _pad_edges(s2, 0)
    s1_dg = _pad_edges(s1, N).reshape(R2D, CH)
    s2_dg = _pad_edges(s2, N).reshape(R2D, CH)
    d1_p = _pad_edges(d1, N)
    d2_p = _pad_edges(d2, N)

    ones_c = jnp.ones((CH,), jnp.float32)
    zeros_deg = jnp.zeros((DPT,), jnp.float32)
    zrows = jnp.zeros((RPT, D), jnp.float32)

    dego1, degi1, dego2, degi2 = _sc_degrees(
        s1_dg, d1_p.reshape(R2D, CH), s2_dg, d2_p.reshape(R2D, CH),
        ones_c, zeros_deg)

    c1 = _gcn_encode(feat1, s1_mm, d1_p, dego1, degi1, W1, b1, W2, b2, zrows)
    c2 = _gcn_encode(feat2, s2_mm, d2_p, dego2, degi2, W1, b1, W2, b2, zrows)

    x21 = _tc_mlp(c1, Wm1, bm1.reshape(1, D), Wm2, bm2.reshape(1, D))
    x22 = _tc_mlp(c2, Wm1, bm1.reshape(1, D), Wm2, bm2.reshape(1, D))
    return (c1, c2, x21, x22)


# speedup vs baseline: 1.7242x; 1.7242x over previous
import functools

import jax
import jax.numpy as jnp
from jax import lax
from jax.experimental import pallas as pl
from jax.experimental.pallas import tpu as pltpu
from jax.experimental.pallas import tpu_sc as plsc

N = 10000
E = 320000
D = 128

NC = 2
NS = 16
NW = NC * NS
CH = 128
R2D = 2560
EPAD = R2D * CH
RPC = R2D // NW
CHS = 80
ESW = E // NW
NCHS = ESW // CHS
NPAD = 10240
RPT = NPAD // NS
DPT = NPAD // NS

_sc_mesh = plsc.VectorSubcoreMesh(core_axis_name="c", subcore_axis_name="s")


@functools.partial(
    pl.kernel,
    out_type=[jax.ShapeDtypeStruct((NC * NPAD,), jnp.float32)] * 4,
    mesh=_sc_mesh,
    scratch_types=[
        pltpu.VMEM((RPC, CH), jnp.int32),
        pltpu.VMEM((RPC, CH), jnp.int32),
        pltpu.VMEM((RPC, CH), jnp.int32),
        pltpu.VMEM((RPC, CH), jnp.int32),
        pltpu.VMEM((CH,), jnp.float32),
        pltpu.VMEM_SHARED((NPAD,), jnp.float32),
        pltpu.VMEM_SHARED((NPAD,), jnp.float32),
        pltpu.VMEM_SHARED((NPAD,), jnp.float32),
        pltpu.VMEM_SHARED((NPAD,), jnp.float32),
        pltpu.SemaphoreType.DMA,
        pltpu.SemaphoreType.DMA,
        pltpu.SemaphoreType.DMA,
        pltpu.SemaphoreType.DMA,
    ],
)
def _sc_degrees(s1_hbm, d1_hbm, s2_hbm, d2_hbm, ones_hbm, zeros_hbm,
                o1_hbm, i1_hbm, o2_hbm, i2_hbm,
                s1_v, d1_v, s2_v, d2_v, ones_v,
                o1_sh, i1_sh, o2_sh, i2_sh,
                sem0, sem1, sem2, sem3):
    cid = lax.axis_index("c")
    sid = lax.axis_index("s")
    wid = cid * NS + sid
    idx_v = (s1_v, d1_v, s2_v, d2_v)
    idx_hbm = (s1_hbm, d1_hbm, s2_hbm, d2_hbm)
    sh = (o1_sh, i1_sh, o2_sh, i2_sh)
    out = (o1_hbm, i1_hbm, o2_hbm, i2_hbm)
    sems = (sem0, sem1, sem2, sem3)

    pltpu.sync_copy(ones_hbm, ones_v)
    for k in range(4):
        pltpu.sync_copy(idx_hbm[k].at[pl.ds(wid * RPC, RPC)], idx_v[k])
        pltpu.sync_copy(zeros_hbm, sh[k].at[pl.ds(sid * DPT, DPT)])
    plsc.subcore_barrier()

    for k in range(4):
        pltpu.async_copy(ones_v, sh[k].at[idx_v[k].at[0]], sems[k], add=True)

    def body(j, carry):
        for k in range(4):
            pltpu.make_async_copy(ones_v, sh[k].at[idx_v[k].at[j - 1]],
                                  sems[k]).wait()
            pltpu.async_copy(ones_v, sh[k].at[idx_v[k].at[j]], sems[k],
                             add=True)
        return carry

    lax.fori_loop(1, RPC, body, 0)
    for k in range(4):
        pltpu.make_async_copy(ones_v, sh[k].at[idx_v[k].at[RPC - 1]],
                              sems[k]).wait()
    plsc.subcore_barrier()
    for k in range(4):
        pltpu.sync_copy(sh[k].at[pl.ds(sid * DPT, DPT)],
                        out[k].at[pl.ds(cid * NPAD + sid * DPT, DPT)])


@functools.partial(
    pl.kernel,
    out_type=jax.ShapeDtypeStruct((NC, NPAD, D), jnp.float32),
    mesh=_sc_mesh,
    scratch_types=[
        pltpu.VMEM((CHS,), jnp.int32),
        pltpu.VMEM((CHS,), jnp.int32),
        pltpu.VMEM((CHS, D), jnp.float32),
        pltpu.VMEM_SHARED((NPAD, D), jnp.float32),
        pltpu.SemaphoreType.DMA,
    ],
)
def _sc_spmm(h_hbm, src_hbm, dst_hbm, zrows_hbm, out_hbm,
             sidx_v, didx_v, rows_v, acc_sh, sem):
    cid = lax.axis_index("c")
    sid = lax.axis_index("s")
    wid = cid * NS + sid
    base = wid * ESW

    pltpu.sync_copy(zrows_hbm, acc_sh.at[pl.ds(sid * RPT, RPT)])
    plsc.subcore_barrier()

    def body(i, carry):
        off = base + i * CHS
        pltpu.sync_copy(src_hbm.at[pl.ds(off, CHS)], sidx_v)
        pltpu.sync_copy(dst_hbm.at[pl.ds(off, CHS)], didx_v)
        pltpu.async_copy(h_hbm.at[sidx_v], rows_v, sem).wait()
        pltpu.sync_copy(rows_v, acc_sh.at[didx_v], add=True)
        return carry

    lax.fori_loop(0, NCHS, body, 0)
    plsc.subcore_barrier()
    pltpu.sync_copy(acc_sh.at[pl.ds(sid * RPT, RPT)],
                    out_hbm.at[cid, pl.ds(sid * RPT, RPT)])


_BN = 2000


def _prep_body(feat_ref, doa_ref, dob_ref, dia_ref, dib_ref,
               h0_ref, ns_ref, nd_ref):
    dego = doa_ref[...] + dob_ref[...]
    degi = dia_ref[...] + dib_ref[...]
    ns = lax.rsqrt(jnp.where(dego > 0, dego, 1.0))
    nd = lax.rsqrt(jnp.where(degi > 0, degi, 1.0))
    ns_ref[...] = ns
    nd_ref[...] = nd
    h0_ref[...] = feat_ref[...] * ns


def _tc_prep(feat, doa, dob, dia, dib):
    grid = (N // _BN,)
    row = pl.BlockSpec((_BN, D), lambda i: (i, 0))
    col = pl.BlockSpec((_BN, 1), lambda i: (i, 0))
    return pl.pallas_call(
        _prep_body,
        grid=grid,
        in_specs=[row, col, col, col, col],
        out_specs=[row, col, col],
        out_shape=[
            jax.ShapeDtypeStruct((N, D), jnp.float32),
            jax.ShapeDtypeStruct((N, 1), jnp.float32),
            jax.ShapeDtypeStruct((N, 1), jnp.float32),
        ],
    )(feat, doa, dob, dia, dib)


def _mm_body(aa_ref, ab_ref, nd_ref, so_ref, w_ref, b_ref, y_ref):
    x = (aa_ref[0] + ab_ref[0]) * nd_ref[...]
    y = jnp.dot(x, w_ref[...], preferred_element_type=jnp.float32)
    y = jnp.maximum(y + b_ref[...], 0.0)
    y_ref[...] = y * so_ref[...]


def _tc_mm(aggp, nd, so, w, b):
    grid = (N // _BN,)
    part_a = pl.BlockSpec((1, _BN, D), lambda i: (0, i, 0))
    part_b = pl.BlockSpec((1, _BN, D), lambda i: (1, i, 0))
    row = pl.BlockSpec((_BN, D), lambda i: (i, 0))
    col = pl.BlockSpec((_BN, 1), lambda i: (i, 0))
    full = pl.BlockSpec((D, D), lambda i: (0, 0))
    vec = pl.BlockSpec((1, D), lambda i: (0, 0))
    return pl.pallas_call(
        _mm_body,
        grid=grid,
        in_specs=[part_a, part_b, col, col, full, vec],
        out_specs=row,
        out_shape=jax.ShapeDtypeStruct((N, D), jnp.float32),
    )(aggp, aggp, nd, so, w, b)


def _mlp_body(c_ref, w1_ref, b1_ref, w2_ref, b2_ref, y_ref):
    t = jnp.dot(c_ref[...], w1_ref[...], preferred_element_type=jnp.float32)
    t = jnp.maximum(t + b1_ref[...], 0.0)
    y = jnp.dot(t, w2_ref[...], preferred_element_type=jnp.float32)
    y_ref[...] = jnp.maximum(y + b2_ref[...], 0.0)


def _tc_mlp(c, w1, b1, w2, b2):
    grid = (N // _BN,)
    row = pl.BlockSpec((_BN, D), lambda i: (i, 0))
    full = pl.BlockSpec((D, D), lambda i: (0, 0))
    vec = pl.BlockSpec((1, D), lambda i: (0, 0))
    return pl.pallas_call(
        _mlp_body,
        grid=grid,
        in_specs=[row, full, vec, full, vec],
        out_specs=row,
        out_shape=jax.ShapeDtypeStruct((N, D), jnp.float32),
    )(c, w1, b1, w2, b2)


def _pad_edges(idx, pad_value):
    pad = jnp.full((EPAD - E,), pad_value, jnp.int32)
    return jnp.concatenate([idx, pad])


def _gcn_encode(feat, src_mm, dst_mm, dego, degi, W1, b1, W2, b2, zrows):
    doa = dego[:N].reshape(N, 1)
    dob = dego[NPAD:NPAD + N].reshape(N, 1)
    dia = degi[:N].reshape(N, 1)
    dib = degi[NPAD:NPAD + N].reshape(N, 1)

    h0, ns, nd = _tc_prep(feat, doa, dob, dia, dib)

    agg1 = _sc_spmm(h0, src_mm, dst_mm, zrows)
    h1 = _tc_mm(agg1, nd, ns, W1, b1.reshape(1, D))

    agg2 = _sc_spmm(h1, src_mm, dst_mm, zrows)
    ones_n = jnp.ones((N, 1), jnp.float32)
    c = _tc_mm(agg2, nd, ones_n, W2, b2.reshape(1, D))
    return c


def kernel(feat1, feat2, edge_index1, edge_index2, W1, b1, W2, b2,
           Wm1, bm1, Wm2, bm2):
    s1, d1 = edge_index1[0], edge_index1[1]
    s2, d2 = edge_index2[0], edge_index2[1]
    s1_dg = _pad_edges(s1, N).reshape(R2D, CH)
    s2_dg = _pad_edges(s2, N).reshape(R2D, CH)
    d1_p = _pad_edges(d1, N)
    d2_p = _pad_edges(d2, N)

    ones_c = jnp.ones((CH,), jnp.float32)
    zeros_deg = jnp.zeros((DPT,), jnp.float32)
    zrows = jnp.zeros((RPT, D), jnp.float32)

    dego1, degi1, dego2, degi2 = _sc_degrees(
        s1_dg, d1_p.reshape(R2D, CH), s2_dg, d2_p.reshape(R2D, CH),
        ones_c, zeros_deg)

    c1 = _gcn_encode(feat1, s1, d1, dego1, degi1, W1, b1, W2, b2, zrows)
    c2 = _gcn_encode(feat2, s2, d2, dego2, degi2, W1, b1, W2, b2, zrows)

    x21 = _tc_mlp(c1, Wm1, bm1.reshape(1, D), Wm2, bm2.reshape(1, D))
    x22 = _tc_mlp(c2, Wm1, bm1.reshape(1, D), Wm2, bm2.reshape(1, D))
    return (c1, c2, x21, x22)
